# baseline (speedup 1.0000x reference)
import functools

import jax
import jax.numpy as jnp
from jax import lax
from jax.experimental import pallas as pl
from jax.experimental.pallas import tpu as pltpu
from jax.experimental.pallas import tpu_sc as plsc

VOCAB = 1000000
DIM = 64
BATCH = 16384
HIST = 50

NC, NS = 2, 16
DPC = DIM // NC
CB = BATCH // 4
NCB = BATCH // CB

_mesh = plsc.VectorSubcoreMesh(core_axis_name="c", subcore_axis_name="s")


@functools.partial(
    pl.kernel,
    out_type=jax.ShapeDtypeStruct((HIST, DIM, BATCH), jnp.float32),
    mesh=_mesh,
    scratch_types=[
        pltpu.VMEM_SHARED((VOCAB,), jnp.float32),
        pltpu.VMEM((5 * CB,), jnp.int32),
        pltpu.VMEM((4 * CB,), jnp.float32),
        pltpu.SemaphoreType.DMA,
        pltpu.SemaphoreType.DMA,
        pltpu.SemaphoreType.DMA,
    ],
)
def _embed_kernel(tok_hbm, table_hbm, out_hbm, row_sh, tokb, gbuf,
                  tsem, gsem, osem):
    c = lax.axis_index("c")
    s = lax.axis_index("s")
    NQ = HIST * NCB
    nu = jnp.where(s < NQ - NS * (NQ // NS), NQ // NS + 1, NQ // NS)

    def tok_src(u):
        q = s + NS * u
        h = q // NCB
        cb = q % NCB
        return tok_hbm.at[pl.ds(h * BATCH + cb * CB, CB)]

    def tok_dst(u):
        return tokb.at[pl.ds((u % 5) * CB, CB)]

    def gslot(u):
        return gbuf.at[pl.ds((u % 4) * CB, CB)]

    def out_dst(u, d):
        q = s + NS * u
        h = q // NCB
        cb = q % NCB
        return out_hbm.at[h, d, pl.ds(cb * CB, CB)]

    def dstep(dloc, carry):
        d = c * DPC + dloc
        pltpu.sync_copy(tok_src(0), tok_dst(0))
        plsc.subcore_barrier()

        @pl.when(s == 0)
        def _load_row():
            pltpu.sync_copy(table_hbm.at[d], row_sh)

        plsc.subcore_barrier()
        @pl.when(dloc > 0)
        def _drain_prev_final():
            pltpu.make_async_copy(
                gslot(nu - 1), out_dst(nu - 1, d - 1), osem
            ).wait()

        pltpu.async_copy(row_sh.at[tok_dst(0)], gslot(0), gsem)
        pltpu.async_copy(row_sh.at[tok_dst(0)], gslot(1), gsem)
        pltpu.async_copy(row_sh.at[tok_dst(0)], gslot(2), gsem)

        def ustep(u, carry):
            pltpu.make_async_copy(row_sh.at[tok_dst(0)], gslot(u), gsem).wait()

            @pl.when(u >= 1)
            def _free_gslot():
                pltpu.make_async_copy(gslot(u - 1), out_dst(u - 1, d), osem).wait()

            @pl.when(u + 3 < nu)
            def _next_gather():
                pltpu.async_copy(row_sh.at[tok_dst(0)], gslot(u + 3), gsem)

            pltpu.async_copy(gslot(u), out_dst(u, d), osem)
            return carry

        lax.fori_loop(0, nu, ustep, 0)
        return carry

    lax.fori_loop(0, DPC, dstep, 0)
    pltpu.make_async_copy(
        gbuf.at[pl.ds(((nu - 1) % 4) * CB, CB)],
        out_hbm.at[0, 0, pl.ds(0, CB)],
        osem,
    ).wait()


def kernel(tokens, token_embedding):
    tok_flat = tokens.T.reshape(HIST * BATCH)
    out_t = _embed_kernel(tok_flat, token_embedding.T)
    return jnp.transpose(out_t, (2, 0, 1))


# speedup vs baseline: 1.0467x; 1.0467x over previous
import functools

import jax
import jax.numpy as jnp
from jax import lax
from jax.experimental import pallas as pl
from jax.experimental.pallas import tpu as pltpu
from jax.experimental.pallas import tpu_sc as plsc

VOCAB = 1000000
DIM = 64
BATCH = 16384
HIST = 50

NC, NS = 2, 16
DPC = DIM // NC
CB = BATCH // 4
NCB = BATCH // CB

_mesh = plsc.VectorSubcoreMesh(core_axis_name="c", subcore_axis_name="s")


@functools.partial(
    pl.kernel,
    out_type=jax.ShapeDtypeStruct((HIST, DIM, BATCH), jnp.float32),
    mesh=_mesh,
    scratch_types=[
        pltpu.VMEM_SHARED((VOCAB,), jnp.float32),
        pltpu.VMEM((5 * CB,), jnp.int32),
        pltpu.VMEM((4 * CB,), jnp.float32),
        pltpu.SemaphoreType.DMA,
        pltpu.SemaphoreType.DMA,
        pltpu.SemaphoreType.DMA,
    ],
)
def _embed_kernel(tok_hbm, table_hbm, out_hbm, row_sh, tokb, gbuf,
                  tsem, gsem, osem):
    c = lax.axis_index("c")
    s = lax.axis_index("s")
    NQ = HIST * NCB
    nu = jnp.where(s < NQ - NS * (NQ // NS), NQ // NS + 1, NQ // NS)

    def tok_src(u):
        q = s + NS * u
        h = q // NCB
        cb = q % NCB
        return tok_hbm.at[pl.ds(h * BATCH + cb * CB, CB)]

    def tok_dst(u):
        return tokb.at[pl.ds((u % 5) * CB, CB)]

    def gslot(u):
        return gbuf.at[pl.ds((u % 4) * CB, CB)]

    def out_dst(u, d):
        q = s + NS * u
        h = q // NCB
        cb = q % NCB
        return out_hbm.at[h, d, pl.ds(cb * CB, CB)]

    def dstep(dloc, carry):
        d = c * DPC + dloc
        pltpu.async_copy(tok_src(0), tok_dst(0), tsem)
        pltpu.async_copy(tok_src(1), tok_dst(1), tsem)
        pltpu.async_copy(tok_src(2), tok_dst(2), tsem)
        pltpu.async_copy(tok_src(3), tok_dst(3), tsem)
        plsc.subcore_barrier()

        @pl.when(s == 0)
        def _load_row():
            pltpu.sync_copy(table_hbm.at[d], row_sh)

        plsc.subcore_barrier()
        @pl.when(dloc > 0)
        def _drain_prev_final():
            pltpu.make_async_copy(
                gslot(nu - 1), out_dst(nu - 1, d - 1), osem
            ).wait()

        pltpu.make_async_copy(tok_src(0), tok_dst(0), tsem).wait()
        pltpu.async_copy(row_sh.at[tok_dst(0)], gslot(0), gsem)
        pltpu.make_async_copy(tok_src(1), tok_dst(1), tsem).wait()
        pltpu.async_copy(row_sh.at[tok_dst(1)], gslot(1), gsem)
        pltpu.make_async_copy(tok_src(2), tok_dst(2), tsem).wait()
        pltpu.async_copy(row_sh.at[tok_dst(2)], gslot(2), gsem)

        def ustep(u, carry):
            @pl.when(u + 4 < nu)
            def _prefetch_tok():
                pltpu.async_copy(tok_src(u + 4), tok_dst(u + 4), tsem)

            pltpu.make_async_copy(row_sh.at[tok_dst(u)], gslot(u), gsem).wait()

            @pl.when(u >= 1)
            def _free_gslot():
                pltpu.make_async_copy(gslot(u - 1), out_dst(u - 1, d), osem).wait()

            @pl.when(u + 3 < nu)
            def _next_gather():
                pltpu.make_async_copy(tok_src(u + 3), tok_dst(u + 3), tsem).wait()
                pltpu.async_copy(row_sh.at[tok_dst(u + 3)], gslot(u + 3), gsem)

            pltpu.async_copy(gslot(u), out_dst(u, d), osem)
            return carry

        lax.fori_loop(0, nu, ustep, 0)
        return carry

    lax.fori_loop(0, DPC, dstep, 0)
    pltpu.make_async_copy(
        gbuf.at[pl.ds(((nu - 1) % 4) * CB, CB)],
        out_hbm.at[0, 0, pl.ds(0, CB)],
        osem,
    ).wait()


def kernel(tokens, token_embedding):
    tok_flat = tokens.T.reshape(HIST * BATCH)
    out_t = _embed_kernel(tok_flat, token_embedding.T)
    return jnp.transpose(out_t, (2, 0, 1))
